# baseline (speedup 1.0000x reference)
import math

import jax
import jax.numpy as jnp
from jax.experimental import pallas as pl
from jax.experimental.pallas import tpu as pltpu

D = 256
H = 4
DK = D // H
N = 4096
K = 16

NB = 4
BN = N // NB

_HIGHEST = jax.lax.Precision.HIGHEST
_DEFAULT = jax.lax.Precision.DEFAULT


def _lin(x, w, b=None, prec=_HIGHEST):
    out = jax.lax.dot_general(x, w, (((1,), (1,)), ((), ())), precision=prec)
    if b is not None:
        out = out + b
    return out


def _layer_norm(x, g, b, eps=1e-5):
    mu = jnp.mean(x, axis=-1, keepdims=True)
    xc = x - mu
    var = jnp.mean(xc * xc, axis=-1, keepdims=True)
    return xc * jax.lax.rsqrt(var + eps) * g + b


def _fused_kernel(
    h_cell_ref,
    s_ref,
    h_tissue_ref,
    wbuq_ref, bbuq_ref, wbuk_ref, bbuk_ref, wbuv_ref, bbuv_ref,
    buq_ref, bubq_ref, buk_ref, bubk_ref, buv_ref, bubv_ref,
    bufc_w_ref, bufc_b_ref, buln_g_ref, buln_b_ref,
    gbu_w_ref, gbu_b_ref,
    wtdv_ref, btdv_ref, tdv_ref, tdbv_ref,
    tdfc_w_ref, tdfc_b_ref, tdln_g_ref, tdln_b_ref,
    gtd_w_ref, gtd_b_ref,
    out_cell_ref,
    out_tissue_ref,
    w3_ref,
    bkv_ref,
    qc_ref,
    m_ref,
    l_ref,
    acc_ref,
    hc_stash_ref,
    gp_stash_ref,
):
    i = pl.program_id(0)

    @pl.when(i == 0)
    def _init():
        q0 = _lin(h_tissue_ref[...], wbuq_ref[...], bbuq_ref[...])
        qc_ref[...] = (_lin(q0, buq_ref[...], bubq_ref[...])
                       * (1.0 / math.sqrt(DK)))
        w3_ref[:D] = jnp.dot(buk_ref[...], wbuk_ref[...], precision=_HIGHEST)
        w3_ref[D:2 * D] = jnp.dot(buv_ref[...], wbuv_ref[...],
                                  precision=_HIGHEST)
        w3_ref[2 * D:] = gtd_w_ref[:, :D]
        bkv_ref[0:1] = _lin(bbuk_ref[...].reshape(1, D), buk_ref[...],
                            bubk_ref[...])
        bkv_ref[1:2] = _lin(bbuv_ref[...].reshape(1, D), buv_ref[...],
                            bubv_ref[...])
        m_ref[...] = jnp.full((K, H), -1e30, jnp.float32)
        l_ref[...] = jnp.zeros((K, H), jnp.float32)
        acc_ref[...] = jnp.zeros((H, K, DK), jnp.float32)

    hc = h_cell_ref[...]
    hc_stash_ref[pl.ds(i * BN, BN), :] = hc
    kvg = _lin(hc, w3_ref[...], prec=_DEFAULT)
    kc = kvg[:, :D] + bkv_ref[0:1]
    vc = kvg[:, D:2 * D] + bkv_ref[1:2]
    gp_stash_ref[pl.ds(i * BN, BN), :] = kvg[:, 2 * D:]
    mask_t = jnp.transpose(s_ref[pl.ds(i * BN, BN), :]) > 0.1
    qc = qc_ref[...]
    for h in range(H):
        q_h = qc[:, h * DK:(h + 1) * DK]
        k_h = kc[:, h * DK:(h + 1) * DK]
        v_h = vc[:, h * DK:(h + 1) * DK]
        s = jax.lax.dot_general(q_h, k_h, (((1,), (1,)), ((), ())),
                                precision=_DEFAULT)
        s = jnp.where(mask_t, s, -jnp.inf)
        m_old = m_ref[:, h:h + 1]
        m_new = jnp.maximum(
            jnp.maximum(m_old, jnp.max(s, axis=1, keepdims=True)), -1e30)
        alpha = jnp.exp(m_old - m_new)
        p = jnp.exp(s - m_new)
        l_ref[:, h:h + 1] = (l_ref[:, h:h + 1] * alpha
                             + jnp.sum(p, axis=1, keepdims=True))
        pv = jnp.dot(p, v_h, precision=_DEFAULT)
        acc_ref[h, :, :] = acc_ref[h, :, :] * alpha + pv
        m_ref[:, h:h + 1] = m_new

    @pl.when(i == NB - 1)
    def _tail():
        ht = h_tissue_ref[...]
        parts = []
        for h in range(H):
            denom = jnp.maximum(l_ref[:, h:h + 1], 1e-30)
            parts.append(acc_ref[h, :, :] / denom)
        attn = jnp.concatenate(parts, axis=1)
        attn = _lin(attn, bufc_w_ref[...], bufc_b_ref[...])
        attn = _layer_norm(attn, buln_g_ref[...], buln_b_ref[...])
        gate = jax.nn.sigmoid(
            _lin(ht, gbu_w_ref[:, :D])
            + _lin(attn, gbu_w_ref[:, D:])
            + gbu_b_ref[...])
        new_rows = gate * attn + (1.0 - gate) * ht
        ht_upd = jnp.where(l_ref[:, 0:1] > 0.0, new_rows, ht)
        out_tissue_ref[...] = ht_upd

        v0 = _lin(ht_upd, wtdv_ref[...], btdv_ref[...])
        v1 = _lin(v0, tdv_ref[...], tdbv_ref[...])
        table = _lin(v1, tdfc_w_ref[...], tdfc_b_ref[...])
        table = _layer_norm(table, tdln_g_ref[...], tdln_b_ref[...])
        gtab = _lin(table, gtd_w_ref[:, D:])
        both = jnp.concatenate([table, gtab], axis=1)

        s_raw = s_ref[...]
        rowmax = jnp.max(s_raw, axis=1, keepdims=True)
        eq = s_raw == rowmax
        col = jax.lax.broadcasted_iota(jnp.int32, (N, K), 1)
        first = jnp.min(jnp.where(eq, col, K), axis=1, keepdims=True)
        onehot = (col == first).astype(jnp.float32)
        gathered = jnp.dot(onehot, both, precision=_DEFAULT)
        attn_c = gathered[:, :D]
        g2 = gathered[:, D:]
        hc_all = hc_stash_ref[...]
        gate_c = jax.nn.sigmoid(gp_stash_ref[...] + g2 + gtd_b_ref[...])
        out_cell_ref[...] = gate_c * attn_c + (1.0 - gate_c) * hc_all


def _full(shape):
    return pl.BlockSpec(shape, lambda i: tuple(0 for _ in shape))


@jax.jit
def kernel(h_cell, h_tissue, S, params):
    p = params
    bu = p['bu']
    td = p['td']

    w_full = _full((D, D))
    b_full = _full((D,))

    out_cell, out_tissue = pl.pallas_call(
        _fused_kernel,
        grid=(NB,),
        in_specs=[
            pl.BlockSpec((BN, D), lambda i: (i, 0)),
            _full((N, K)),
            _full((K, D)),
            w_full, b_full, w_full, b_full, w_full, b_full,
            w_full, b_full, w_full, b_full, w_full, b_full,
            w_full, b_full, b_full, b_full,
            _full((D, 2 * D)), b_full,
            w_full, b_full, w_full, b_full,
            w_full, b_full, b_full, b_full,
            _full((D, 2 * D)), b_full,
        ],
        out_specs=(
            _full((N, D)),
            _full((K, D)),
        ),
        out_shape=(
            jax.ShapeDtypeStruct((N, D), jnp.float32),
            jax.ShapeDtypeStruct((K, D), jnp.float32),
        ),
        scratch_shapes=[
            pltpu.VMEM((3 * D, D), jnp.float32),
            pltpu.VMEM((2, D), jnp.float32),
            pltpu.VMEM((K, D), jnp.float32),
            pltpu.VMEM((K, H), jnp.float32),
            pltpu.VMEM((K, H), jnp.float32),
            pltpu.VMEM((H, K, DK), jnp.float32),
            pltpu.VMEM((N, D), jnp.float32),
            pltpu.VMEM((N, D), jnp.float32),
        ],
    )(
        h_cell, S, h_tissue,
        p['Wbuq_w'], p['Wbuq_b'], p['Wbuk_w'], p['Wbuk_b'],
        p['Wbuv_w'], p['Wbuv_b'],
        bu['Wq_w'], bu['Wq_b'], bu['Wk_w'], bu['Wk_b'],
        bu['Wv_w'], bu['Wv_b'],
        bu['fc_w'], bu['fc_b'], bu['ln_g'], bu['ln_b'],
        p['gbu_w'], p['gbu_b'],
        p['Wtdv_w'], p['Wtdv_b'], td['Wv_w'], td['Wv_b'],
        td['fc_w'], td['fc_b'], td['ln_g'], td['ln_b'],
        p['gtd_w'], p['gtd_b'],
    )
    return out_cell, out_tissue


# speedup vs baseline: 1.1734x; 1.1734x over previous
import math

import jax
import jax.numpy as jnp
from jax.experimental import pallas as pl
from jax.experimental.pallas import tpu as pltpu

D = 256
H = 4
DK = D // H
N = 4096
K = 16

_HIGHEST = jax.lax.Precision.HIGHEST
_DEFAULT = jax.lax.Precision.DEFAULT


def _lin(x, w, b=None, prec=_HIGHEST):
    out = jax.lax.dot_general(x, w, (((1,), (1,)), ((), ())), precision=prec)
    if b is not None:
        out = out + b
    return out


def _layer_norm(x, g, b, eps=1e-5):
    mu = jnp.mean(x, axis=-1, keepdims=True)
    xc = x - mu
    var = jnp.mean(xc * xc, axis=-1, keepdims=True)
    return xc * jax.lax.rsqrt(var + eps) * g + b


def _fused_kernel(
    h_cell_ref,
    s_ref,
    h_tissue_ref,
    wbuq_ref, bbuq_ref, wbuk_ref, bbuk_ref, wbuv_ref, bbuv_ref,
    buq_ref, bubq_ref, buk_ref, bubk_ref, buv_ref, bubv_ref,
    bufc_w_ref, bufc_b_ref, buln_g_ref, buln_b_ref,
    gbu_w_ref, gbu_b_ref,
    wtdv_ref, btdv_ref, tdv_ref, tdbv_ref,
    tdfc_w_ref, tdfc_b_ref, tdln_g_ref, tdln_b_ref,
    gtd_w_ref, gtd_b_ref,
    out_cell_ref,
    out_tissue_ref,
):
    hc = h_cell_ref[...]
    ht = h_tissue_ref[...]
    s_raw = s_ref[...]

    q0 = _lin(ht, wbuq_ref[...], bbuq_ref[...])
    qc = _lin(q0, buq_ref[...], bubq_ref[...]) * (1.0 / math.sqrt(DK))
    wkc = jnp.dot(buk_ref[...], wbuk_ref[...], precision=_HIGHEST)
    bkc = _lin(bbuk_ref[...].reshape(1, D), buk_ref[...], bubk_ref[...])
    wvc = jnp.dot(buv_ref[...], wbuv_ref[...], precision=_HIGHEST)
    bvc = _lin(bbuv_ref[...].reshape(1, D), buv_ref[...], bubv_ref[...])
    w3 = jnp.concatenate([wkc, wvc, gtd_w_ref[:, :D]], axis=0)
    kvg = _lin(hc, w3, prec=_DEFAULT)
    kc = kvg[:, :D] + bkc
    vc = kvg[:, D:2 * D] + bvc
    gpart = kvg[:, 2 * D:]

    mask_t = jnp.transpose(s_raw) > 0.1
    parts = []
    l0 = None
    for h in range(H):
        q_h = qc[:, h * DK:(h + 1) * DK]
        k_h = kc[:, h * DK:(h + 1) * DK]
        v_h = vc[:, h * DK:(h + 1) * DK]
        s = jax.lax.dot_general(q_h, k_h, (((1,), (1,)), ((), ())),
                                precision=_DEFAULT)
        s = jnp.where(mask_t, s, -jnp.inf)
        m = jnp.maximum(jnp.max(s, axis=1, keepdims=True), -1e30)
        p = jnp.exp(s - m)
        l = jnp.sum(p, axis=1, keepdims=True)
        if h == 0:
            l0 = l
        pv = jnp.dot(p, v_h, precision=_DEFAULT)
        parts.append(pv / jnp.maximum(l, 1e-30))
    attn = jnp.concatenate(parts, axis=1)
    attn = _lin(attn, bufc_w_ref[...], bufc_b_ref[...])
    attn = _layer_norm(attn, buln_g_ref[...], buln_b_ref[...])
    gate = jax.nn.sigmoid(
        _lin(ht, gbu_w_ref[:, :D])
        + _lin(attn, gbu_w_ref[:, D:])
        + gbu_b_ref[...])
    new_rows = gate * attn + (1.0 - gate) * ht
    ht_upd = jnp.where(l0 > 0.0, new_rows, ht)
    out_tissue_ref[...] = ht_upd

    v0 = _lin(ht_upd, wtdv_ref[...], btdv_ref[...])
    v1 = _lin(v0, tdv_ref[...], tdbv_ref[...])
    table = _lin(v1, tdfc_w_ref[...], tdfc_b_ref[...])
    table = _layer_norm(table, tdln_g_ref[...], tdln_b_ref[...])
    gtab = _lin(table, gtd_w_ref[:, D:])

    rowmax = jnp.max(s_raw, axis=1, keepdims=True)
    eq = s_raw == rowmax
    col = jax.lax.broadcasted_iota(jnp.int32, (N, K), 1)
    first = jnp.min(jnp.where(eq, col, K), axis=1, keepdims=True)
    onehot = (col == first).astype(jnp.float32)

    both = jnp.concatenate([table, gtab], axis=1)
    gathered = jnp.dot(onehot, both, precision=_DEFAULT)
    attn_c = gathered[:, :D]
    g2 = gathered[:, D:]
    gate_c = jax.nn.sigmoid(gpart + g2 + gtd_b_ref[...])
    out_cell_ref[...] = gate_c * attn_c + (1.0 - gate_c) * hc


@jax.jit
def kernel(h_cell, h_tissue, S, params):
    p = params
    bu = p['bu']
    td = p['td']

    out_cell, out_tissue = pl.pallas_call(
        _fused_kernel,
        out_shape=(
            jax.ShapeDtypeStruct((N, D), jnp.float32),
            jax.ShapeDtypeStruct((K, D), jnp.float32),
        ),
    )(
        h_cell, S, h_tissue,
        p['Wbuq_w'], p['Wbuq_b'], p['Wbuk_w'], p['Wbuk_b'],
        p['Wbuv_w'], p['Wbuv_b'],
        bu['Wq_w'], bu['Wq_b'], bu['Wk_w'], bu['Wk_b'],
        bu['Wv_w'], bu['Wv_b'],
        bu['fc_w'], bu['fc_b'], bu['ln_g'], bu['ln_b'],
        p['gbu_w'], p['gbu_b'],
        p['Wtdv_w'], p['Wtdv_b'], td['Wv_w'], td['Wv_b'],
        td['fc_w'], td['fc_b'], td['ln_g'], td['ln_b'],
        p['gtd_w'], p['gtd_b'],
    )
    return out_cell, out_tissue
